# Initial kernel scaffold; baseline (speedup 1.0000x reference)
import jax
import jax.numpy as jnp
from jax.experimental import pallas as pl


def kernel(pos, vel, edge_index, edge_w, Wemb, Wse, Wr1, br1, Wr2, Wqs, Wqv, Wss, Wsv, Wr1f, br1f, Wr2f, Wqf, Wof):
    raise NotImplementedError("write your pallas kernel here")


# speedup vs baseline: 12.2802x; 12.2802x over previous
import functools

import jax
import jax.numpy as jnp
from jax import lax
from jax.experimental import pallas as pl
from jax.experimental.pallas import tpu as pltpu
from jax.experimental.pallas import tpu_sc as plsc

_B = 512


def _sc_gather_call(table, idx):
    N, F = table.shape
    assert F == 128
    E = idx.shape[0]
    info = plsc.get_sparse_core_info()
    NW = info.num_cores * info.num_subcores
    per_w = E // NW
    assert E % NW == 0 and per_w % 8 == 0
    CH = 512
    nfull = per_w // CH
    tail = per_w - nfull * CH
    tail_full = tail // 128
    tail_rem = tail % 128
    assert tail_rem % 8 == 0
    mesh = plsc.VectorSubcoreMesh(core_axis_name="c", subcore_axis_name="s")

    @functools.partial(
        pl.kernel, mesh=mesh,
        out_type=jax.ShapeDtypeStruct((E, F), jnp.float32),
        scratch_types=[
            pltpu.VMEM((CH,), jnp.int32),
            pltpu.VMEM((CH, F), jnp.float32),
            pltpu.SemaphoreType.DMA,
        ],
    )
    def gk(table_hbm, idx_hbm, out_hbm, idx_v, rows_v, sem):
        wid = lax.axis_index("s") * info.num_cores + lax.axis_index("c")
        base = wid * per_w

        def do_chunk(off, count):
            pltpu.sync_copy(idx_hbm.at[pl.ds(off, count)],
                            idx_v.at[pl.ds(0, count)])
            nsub = count // 128
            rem = count % 128
            cps = []
            for b in range(nsub):
                cps.append(pltpu.async_copy(
                    table_hbm.at[idx_v.at[pl.ds(b * 128, 128)]],
                    rows_v.at[pl.ds(b * 128, 128)], sem))
            if rem:
                cps.append(pltpu.async_copy(
                    table_hbm.at[idx_v.at[pl.ds(nsub * 128, rem)]],
                    rows_v.at[pl.ds(nsub * 128, rem)], sem))
            for c in cps:
                c.wait()
            pltpu.sync_copy(rows_v.at[pl.ds(0, count)],
                            out_hbm.at[pl.ds(off, count)])

        if nfull:
            def body(g, _):
                do_chunk(base + g * CH, CH)
                return _
            lax.fori_loop(0, nfull, body, None)
        if tail:
            do_chunk(base + nfull * CH, tail)

    return gk(table, idx)


_NP1 = 51200
_NP2 = 50048
_SCH = 2000


def _vtake(x, idx):
    return lax.gather(
        x, idx[:, None],
        lax.GatherDimensionNumbers(
            offset_dims=(), collapsed_slice_dims=(0,), start_index_map=(0,)),
        (1,), mode=lax.GatherScatterMode.PROMISE_IN_BOUNDS)


def _dedup_combine(k, v, iota, op):
    acc = v
    islast = jnp.full((16,), True)
    for sh in range(1, 16):
        pidx = jnp.maximum(iota - sh, 0)
        kp = _vtake(k, pidx)
        vp = _vtake(v, pidx)
        take = (iota >= sh) & (kp == k)
        if op == "max":
            acc = jnp.where(take, jnp.maximum(acc, vp), acc)
        else:
            acc = acc + jnp.where(take, vp, 0.0)
        nidx = jnp.minimum(iota + sh, 15)
        kn = _vtake(k, nidx)
        islast = islast & ~((iota + sh <= 15) & (kn == k))
    return acc, islast


def _sc_segmax_call(dst, logit):
    E = dst.shape[0]
    info = plsc.get_sparse_core_info()
    NW = info.num_cores * info.num_subcores
    per_w = E // NW
    assert E % NW == 0 and per_w % _SCH == 0
    nch = per_w // _SCH
    NV = _SCH // 16
    STR = _NP1 // 16
    mesh = plsc.VectorSubcoreMesh(core_axis_name="c", subcore_axis_name="s")

    @functools.partial(
        pl.kernel, mesh=mesh,
        out_type=jax.ShapeDtypeStruct((2, _NP1), jnp.float32),
        scratch_types=[
            pltpu.VMEM((_NP1,), jnp.float32),
            pltpu.VMEM((_SCH,), jnp.int32),
            pltpu.VMEM((_SCH,), jnp.float32),
            pltpu.VMEM((STR,), jnp.float32),
            pltpu.VMEM((STR,), jnp.float32),
            pltpu.VMEM_SHARED((16, _NP1), jnp.float32),
        ],
    )
    def ka(dst_hbm, lg_hbm, out_hbm, mtab, dstb, lgb, acc, tbuf, shared):
        c = lax.axis_index("c")
        sid = lax.axis_index("s")
        wid = sid * info.num_cores + c
        neg = jnp.full((16,), -3.0e38, jnp.float32)

        def initb(i, carry):
            mtab[pl.ds(i * 16, 16)] = neg
            return carry
        lax.fori_loop(0, _NP1 // 16, initb, 0)

        iota = lax.iota(jnp.int32, 16)
        base = wid * per_w

        def chunkb(g, carry):
            pltpu.sync_copy(dst_hbm.at[pl.ds(base + g * _SCH, _SCH)], dstb)
            pltpu.sync_copy(lg_hbm.at[pl.ds(base + g * _SCH, _SCH)], lgb)

            def vregb(v, carry2):
                k = dstb[pl.ds(v * 16, 16)]
                val = lgb[pl.ds(v * 16, 16)]
                sv, last = _dedup_combine(k, val, iota, "max")
                k2 = jnp.where(last, k, _NP1 - 16 + iota)
                cur = plsc.load_gather(mtab, [k2])
                plsc.store_scatter(mtab, [k2], jnp.maximum(cur, sv))
                return carry2
            lax.fori_loop(0, NV, vregb, 0)
            return carry
        lax.fori_loop(0, nch, chunkb, 0)

        pltpu.sync_copy(mtab, shared.at[sid])
        plsc.subcore_barrier()
        pltpu.sync_copy(shared.at[0, pl.ds(sid * STR, STR)], acc)
        for t in range(1, 16):
            pltpu.sync_copy(shared.at[t, pl.ds(sid * STR, STR)], tbuf)

            def maxb(i, carry, _t=t):
                acc[pl.ds(i * 16, 16)] = jnp.maximum(
                    acc[pl.ds(i * 16, 16)], tbuf[pl.ds(i * 16, 16)])
                return carry
            lax.fori_loop(0, STR // 16, maxb, 0)
        pltpu.sync_copy(acc, out_hbm.at[c, pl.ds(sid * STR, STR)])

    return ka(dst, logit)


def _sc_exden_call(dst, logit, m, want_den):
    E = dst.shape[0]
    info = plsc.get_sparse_core_info()
    NW = info.num_cores * info.num_subcores
    per_w = E // NW
    assert E % NW == 0 and per_w % _SCH == 0
    nch = per_w // _SCH
    NV = _SCH // 16
    STR = _NP1 // 16
    mesh = plsc.VectorSubcoreMesh(core_axis_name="c", subcore_axis_name="s")

    out_type = [jax.ShapeDtypeStruct((E,), jnp.float32)]
    scratch = [
        pltpu.VMEM((_NP1,), jnp.float32),
        pltpu.VMEM((_SCH,), jnp.int32),
        pltpu.VMEM((_SCH,), jnp.float32),
        pltpu.VMEM((_SCH,), jnp.float32),
    ]
    if want_den:
        out_type.append(jax.ShapeDtypeStruct((2, _NP1), jnp.float32))
        scratch += [
            pltpu.VMEM((_NP1,), jnp.float32),
            pltpu.VMEM((STR,), jnp.float32),
            pltpu.VMEM((STR,), jnp.float32),
            pltpu.VMEM_SHARED((16, _NP1), jnp.float32),
        ]

    @functools.partial(pl.kernel, mesh=mesh, out_type=out_type,
                       scratch_types=scratch)
    def kb(dst_hbm, lg_hbm, m_hbm, *refs):
        if want_den:
            (ex_hbm, den_hbm, mfull, dstb, lgb, exb,
             dentab, acc, tbuf, shared) = refs
        else:
            ex_hbm, mfull, dstb, lgb, exb = refs
        c = lax.axis_index("c")
        sid = lax.axis_index("s")
        wid = sid * info.num_cores + c
        pltpu.sync_copy(m_hbm, mfull)
        if want_den:
            zero = jnp.zeros((16,), jnp.float32)

            def initb(i, carry):
                dentab[pl.ds(i * 16, 16)] = zero
                return carry
            lax.fori_loop(0, _NP1 // 16, initb, 0)

        iota = lax.iota(jnp.int32, 16)
        base = wid * per_w

        def chunkb(g, carry):
            pltpu.sync_copy(dst_hbm.at[pl.ds(base + g * _SCH, _SCH)], dstb)
            pltpu.sync_copy(lg_hbm.at[pl.ds(base + g * _SCH, _SCH)], lgb)

            def vregb(v, carry2):
                k = dstb[pl.ds(v * 16, 16)]
                val = lgb[pl.ds(v * 16, 16)]
                mv = plsc.load_gather(mfull, [k])
                e = jnp.exp(val - mv)
                exb[pl.ds(v * 16, 16)] = e
                if want_den:
                    se, last = _dedup_combine(k, e, iota, "add")
                    k2 = jnp.where(last, k, _NP1 - 16 + iota)
                    cur = plsc.load_gather(dentab, [k2])
                    plsc.store_scatter(dentab, [k2], cur + se)
                return carry2
            lax.fori_loop(0, NV, vregb, 0)
            pltpu.sync_copy(exb, ex_hbm.at[pl.ds(base + g * _SCH, _SCH)])
            return carry
        lax.fori_loop(0, nch, chunkb, 0)

        if want_den:
            pltpu.sync_copy(dentab, shared.at[sid])
            plsc.subcore_barrier()
            pltpu.sync_copy(shared.at[0, pl.ds(sid * STR, STR)], acc)
            for t in range(1, 16):
                pltpu.sync_copy(shared.at[t, pl.ds(sid * STR, STR)], tbuf)

                def addb(i, carry, _t=t):
                    acc[pl.ds(i * 16, 16)] = (acc[pl.ds(i * 16, 16)]
                                              + tbuf[pl.ds(i * 16, 16)])
                    return carry
                lax.fori_loop(0, STR // 16, addb, 0)
            pltpu.sync_copy(acc, den_hbm.at[c, pl.ds(sid * STR, STR)])

    return kb(dst, logit, m)


_SLAB = 4096
_SLAB_USE = 4080
_NSLAB = 13


def _sc_scatadd_call(dst2, P, ncols, split_features):
    nrow, _ = dst2.shape
    EP = nrow * 128
    info = plsc.get_sparse_core_info()
    mesh = plsc.VectorSubcoreMesh(core_axis_name="c", subcore_axis_name="s")
    CH = 2048
    if split_features:
        per_t = EP // 16
    else:
        per_t = EP // 32
    assert per_t % CH == 0
    nch = per_t // CH
    STR = _SLAB // 16

    @functools.partial(
        pl.kernel, mesh=mesh,
        out_type=jax.ShapeDtypeStruct((2, _NSLAB * _SLAB, ncols), jnp.float32),
        scratch_types=[
            pltpu.VMEM((CH // 128, 128), jnp.int32),
            pltpu.VMEM((CH // 128, 128), jnp.int32),
            pltpu.VMEM((CH, ncols), jnp.float32),
            pltpu.VMEM((STR, ncols), jnp.float32),
            pltpu.VMEM_SHARED((_SLAB, ncols), jnp.float32),
            pltpu.SemaphoreType.DMA,
        ],
    )
    def kc(dst_hbm, p_hbm, out_hbm, dstb, dstb2, pbuf, zbuf, accs, sem):
        c = lax.axis_index("c")
        sid = lax.axis_index("s")
        zero = jnp.zeros((16,), jnp.float32)
        iota = lax.iota(jnp.int32, 16)

        def zb(i, carry):
            def zc(j, carry2):
                zbuf[i, pl.ds(j * 16, 16)] = zero
                return carry2
            return lax.fori_loop(0, ncols // 16, zc, carry)
        lax.fori_loop(0, STR, zb, 0)
        sbase = pl.multiple_of(sid * STR, 8)

        if split_features:
            base = sid * per_t
            psrc = p_hbm.at[c]
        else:
            base = (sid * info.num_cores + c) * per_t
            psrc = p_hbm

        for slab in range(_NSLAB):
            lo = slab * _SLAB_USE
            pltpu.sync_copy(zbuf, accs.at[pl.ds(sbase, STR), :])
            plsc.subcore_barrier()

            def chunkb(g, carry, _lo=lo):
                row0 = pl.multiple_of((base + g * CH) // 128, 8)
                eoff = pl.multiple_of(base + g * CH, 8)
                pltpu.sync_copy(dst_hbm.at[pl.ds(row0, CH // 128), :], dstb)
                pltpu.sync_copy(psrc.at[pl.ds(eoff, CH), :], pbuf)
                for b in range(CH // 128):
                    for j in range(8):
                        v = dstb[b, pl.ds(j * 16, 16)] - _lo
                        ok = (v >= 0) & (v < _SLAB_USE)
                        dstb2[b, pl.ds(j * 16, 16)] = jnp.where(
                            ok, v, _SLAB_USE + iota)
                cps = []
                for b in range(CH // 128):
                    cps.append(pltpu.async_copy(
                        pbuf.at[pl.ds(b * 128, 128), :],
                        accs.at[dstb2.at[b]], sem, add=True))
                for cp in cps:
                    cp.wait()
                return carry
            lax.fori_loop(0, nch, chunkb, 0)

            plsc.subcore_barrier()
            pltpu.sync_copy(
                accs.at[pl.ds(sbase, STR), :],
                out_hbm.at[c, pl.ds(slab * _SLAB + sbase, STR), :])
            plsc.subcore_barrier()

    return kc(dst2, P)


def _unslab(part):
    F = part.shape[-1]
    return part.reshape(_NSLAB, _SLAB, F)[:, :_SLAB_USE].reshape(-1, F)


def _tc_scale_call(msg, ex, EP):
    E, F = msg.shape
    B = _B if E % _B == 0 else E
    grid = E // B

    def body(ms_ref, ex_ref, p_ref):
        p = ms_ref[...] * ex_ref[...][:, None]
        p_ref[...] = jnp.stack([p[:, :32], p[:, 32:64]], axis=0)

    return pl.pallas_call(
        body,
        grid=(grid,),
        in_specs=[pl.BlockSpec((B, F), lambda i: (i, 0)),
                  pl.BlockSpec((B,), lambda i: (i,))],
        out_specs=pl.BlockSpec((2, B, 32), lambda i: (0, i, 0)),
        out_shape=jax.ShapeDtypeStruct((2, EP, 32), jnp.float32),
        interpret=False,
    )(msg, ex)


def _tc_expand_ex_call(ex, EP):
    E = ex.shape[0]
    B = _B if E % _B == 0 else E
    grid = E // B

    def body(ex_ref, p_ref):
        e = ex_ref[...][:, None]
        p_ref[...] = jnp.concatenate(
            [e, jnp.zeros((B, 15), jnp.float32)], axis=1)

    return pl.pallas_call(
        body,
        grid=(grid,),
        in_specs=[pl.BlockSpec((B,), lambda i: (i,))],
        out_specs=pl.BlockSpec((B, 16), lambda i: (i, 0)),
        out_shape=jax.ShapeDtypeStruct((EP, 16), jnp.float32),
        interpret=False,
    )(ex)


def _tc_scale_final_call(vvf, ex, EP):
    E = vvf.shape[0]
    B = _B if E % _B == 0 else E
    grid = E // B

    def body(vv_ref, ex_ref, p_ref):
        e = ex_ref[...][:, None]
        p_ref[...] = jnp.concatenate(
            [vv_ref[...] * e, e, jnp.zeros((B, 9), jnp.float32)], axis=1)

    return pl.pallas_call(
        body,
        grid=(grid,),
        in_specs=[pl.BlockSpec((B, 6), lambda i: (i, 0)),
                  pl.BlockSpec((B,), lambda i: (i,))],
        out_specs=pl.BlockSpec((B, 16), lambda i: (i, 0)),
        out_shape=jax.ShapeDtypeStruct((EP, 16), jnp.float32),
        interpret=False,
    )(vvf, ex)


def _edge_layer_call(geomT, gsv, gq, A1, b1, WksT, W3T):
    E = geomT.shape[1]
    B = _B if E % _B == 0 else E
    grid = E // B

    def body(geom_ref, gsv_ref, gq_ref, a1_ref, b1_ref, wks_ref,
             w3_ref, ms_ref, lg_ref):
        geom = geom_ref[...]
        r = geom[0:1]
        ew = geom[1:2]
        rhat = geom[2:5]
        a1 = a1_ref[...]
        h = jnp.maximum(a1[:, 0:1] * r + a1[:, 1:2] * ew + b1_ref[...], 0.0)
        gsvb = gsv_ref[...].T
        gs = gsvb[0:16]
        gV = gsvb[16:64]
        gV3 = gV.reshape(16, 3, B)
        vdotr = jnp.sum(gV3 * rhat[None, :, :], axis=1)
        g32 = jnp.concatenate([gs, vdotr], axis=0)
        U1 = (h[:, None, :] * g32[None, :, :]).reshape(512, B)
        kv = jnp.dot(wks_ref[...], U1, preferred_element_type=jnp.float32)
        k_s = kv[0:16]
        v_coef = kv[16:32]
        U2 = (h[:, None, :] * gV[None, :, :]).reshape(768, B)
        vv = jnp.dot(w3_ref[...], U2, preferred_element_type=jnp.float32)
        vvec = (vv.reshape(16, 3, B)
                + v_coef[:, None, :] * rhat[None, :, :]).reshape(48, B)
        gq = gq_ref[...].T
        logit = (jnp.sum(gq[0:16] * k_s, axis=0, keepdims=True)
                 + jnp.sum(gq[16:64] * vvec, axis=0, keepdims=True)) * 0.25
        ms_ref[...] = jnp.concatenate([k_s, vvec], axis=0).T
        lg_ref[...] = logit

    ebs = lambda rows: pl.BlockSpec((rows, B), lambda i: (0, i))
    rbs = lambda cols: pl.BlockSpec((B, cols), lambda i: (i, 0))
    wbs = lambda shape: pl.BlockSpec(shape, lambda i: (0, 0))
    return pl.pallas_call(
        body,
        grid=(grid,),
        in_specs=[ebs(8), rbs(128), rbs(128), wbs((16, 2)),
                  wbs((16, 1)), wbs((32, 512)), wbs((48, 768))],
        out_specs=[rbs(64), ebs(1)],
        out_shape=[jax.ShapeDtypeStruct((E, 64), jnp.float32),
                   jax.ShapeDtypeStruct((1, E), jnp.float32)],
        interpret=False,
    )(geomT, gsv, gq, A1, b1, WksT, W3T)


def _edge_final_call(geomT, gsv, gq, A1, b1, W0fT, W1fT):
    E = geomT.shape[1]
    B = _B if E % _B == 0 else E
    grid = E // B

    def body(geom_ref, gsv_ref, gq_ref, a1_ref, b1_ref, w0_ref,
             w1_ref, vv_ref, lg_ref):
        geom = geom_ref[...]
        r = geom[0:1]
        ew = geom[1:2]
        rhat = geom[2:5]
        a1 = a1_ref[...]
        h = jnp.maximum(a1[:, 0:1] * r + a1[:, 1:2] * ew + b1_ref[...], 0.0)
        gsvb = gsv_ref[...].T
        gs = gsvb[0:16]
        gV = gsvb[16:64]
        U1 = (h[:, None, :] * gs[None, :, :]).reshape(256, B)
        coef = jnp.dot(w0_ref[...], U1, preferred_element_type=jnp.float32)
        U2 = (h[:, None, :] * gV[None, :, :]).reshape(768, B)
        vv = jnp.dot(w1_ref[...], U2, preferred_element_type=jnp.float32)
        vvec = (vv.reshape(2, 3, B)
                + coef[:, None, :] * rhat[None, :, :]).reshape(6, B)
        gq = gq_ref[...].T
        logit = jnp.sum(gq[0:6] * vvec, axis=0, keepdims=True) * (1.0 / jnp.sqrt(2.0))
        vv_ref[...] = vvec.T
        lg_ref[...] = logit

    ebs = lambda rows: pl.BlockSpec((rows, B), lambda i: (0, i))
    rbs = lambda cols: pl.BlockSpec((B, cols), lambda i: (i, 0))
    wbs = lambda shape: pl.BlockSpec(shape, lambda i: (0, 0))
    return pl.pallas_call(
        body,
        grid=(grid,),
        in_specs=[ebs(8), rbs(128), rbs(128), wbs((16, 2)),
                  wbs((16, 1)), wbs((2, 256)), wbs((6, 768))],
        out_specs=[rbs(6), ebs(1)],
        out_shape=[jax.ShapeDtypeStruct((E, 6), jnp.float32),
                   jax.ShapeDtypeStruct((1, E), jnp.float32)],
        interpret=False,
    )(geomT, gsv, gq, A1, b1, W0fT, W1fT)


def _prep_layer_weights(Wr2_l):
    Wp = Wr2_l.reshape(16, 4, 16, 16)
    ks_s = Wp[:, 0].transpose(0, 2, 1)
    ks_r = Wp[:, 1].transpose(0, 2, 1)
    vc_s = Wp[:, 2].transpose(0, 2, 1)
    T = jnp.zeros((16, 32, 32), jnp.float32)
    T = T.at[:, :16, :16].set(ks_s)
    T = T.at[:, 16:, :16].set(ks_r)
    T = T.at[:, :16, 16:].set(vc_s)
    WksT = T.reshape(512, 32).T
    eye3 = jnp.eye(3, dtype=jnp.float32)
    T3 = jnp.einsum('hij,de->hjdie', Wp[:, 3], eye3)
    W3T = T3.reshape(768, 48).T
    return WksT, W3T


def _prep_final_weights(Wr2f):
    Wpf = Wr2f.reshape(16, 2, 2, 16)
    W0fT = Wpf[:, 0].transpose(1, 0, 2).reshape(2, 256)
    eye3 = jnp.eye(3, dtype=jnp.float32)
    T3 = jnp.einsum('hoc,de->hcdoe', Wpf[:, 1], eye3)
    W1fT = T3.reshape(768, 6).T
    return W0fT, W1fT


def kernel(pos, vel, edge_index, edge_w, Wemb, Wse, Wr1, br1, Wr2, Wqs, Wqv,
           Wss, Wsv, Wr1f, br1f, Wr2f, Wqf, Wof):
    N = pos.shape[0]
    E = edge_index.shape[1]
    src = edge_index[0]
    dst = edge_index[1]
    EP = ((E + 65535) // 65536) * 65536
    dst_pad = jnp.pad(dst, (0, EP - E), constant_values=N)
    dst2 = dst_pad.reshape(EP // 128, 128)

    posp = jnp.pad(pos, ((0, 0), (0, 125)))
    ps = _sc_gather_call(posp, src)[:, :3]
    pd = _sc_gather_call(posp, dst)[:, :3]
    rel = pd - ps
    rn = jnp.linalg.norm(rel, axis=-1, keepdims=True) + 1e-8
    rhat = rel / rn
    geomT = jnp.concatenate(
        [rn.T, edge_w.T, rhat.T, jnp.zeros((3, E), jnp.float32)], axis=0)

    sp = jnp.linalg.norm(vel, axis=-1, keepdims=True)
    s = sp * Wse[0][None, :]
    V = vel[:, None, :] * Wemb[0][None, :, None]

    L = Wr1.shape[0]
    for l in range(L):
        WksT, W3T = _prep_layer_weights(Wr2[l])
        A1 = Wr1[l].T
        b1 = br1[l][:, None]
        q_s = s @ Wqs[l]
        q_v = jnp.einsum('ncd,ci->nid', V, Wqv[l]).reshape(N, 48)
        Vf = V.reshape(N, 48)
        gsv = _sc_gather_call(
            jnp.pad(jnp.concatenate([s, Vf], axis=-1), ((0, 0), (0, 64))), src)
        gq = _sc_gather_call(
            jnp.pad(jnp.concatenate([q_s, q_v], axis=-1), ((0, 0), (0, 64))), dst)
        msg, lgT = _edge_layer_call(geomT, gsv, gq, A1, b1, WksT, W3T)
        logit = lgT[0]
        m = jax.ops.segment_max(logit, dst, num_segments=N)
        ex = jnp.exp(logit - m[dst])
        den = jax.ops.segment_sum(ex, dst, num_segments=N) + 1e-9
        num = jax.ops.segment_sum(ex[:, None] * msg, dst, num_segments=N)
        agg = num / den[:, None]
        agg_s = agg[:, :16]
        agg_v = agg[:, 16:]
        s = jax.nn.relu(s + agg_s @ Wss[l])
        V = V + jnp.einsum('ncd,ci->nid', agg_v.reshape(N, 16, 3), Wsv[l])
        vn = jnp.linalg.norm(V, axis=-1, keepdims=True) + 1e-8
        V = V * jax.nn.sigmoid(vn)

    W0fT, W1fT = _prep_final_weights(Wr2f)
    A1f = Wr1f.T
    b1f = br1f[:, None]
    q_vf = jnp.einsum('ncd,co->nod', V, Wqf).reshape(N, 6)
    gsvf = _sc_gather_call(
        jnp.pad(jnp.concatenate([s, V.reshape(N, 48)], axis=-1),
                ((0, 0), (0, 64))), src)
    gqf = _sc_gather_call(jnp.pad(q_vf, ((0, 0), (0, 122))), dst)
    vvf, lgT = _edge_final_call(geomT, gsvf, gqf, A1f, b1f, W0fT, W1fT)
    logit = lgT[0]
    m = jax.ops.segment_max(logit, dst, num_segments=N)
    exf = jnp.exp(logit - m[dst])
    den = jax.ops.segment_sum(exf, dst, num_segments=N) + 1e-9
    num = jax.ops.segment_sum(exf[:, None] * vvf, dst, num_segments=N)
    agg = num / den[:, None]
    out = jnp.einsum('nod,op->npd', agg.reshape(N, 2, 3), Wof)
    return out


# speedup vs baseline: 12.2849x; 1.0004x over previous
import functools

import jax
import jax.numpy as jnp
from jax import lax
from jax.experimental import pallas as pl
from jax.experimental.pallas import tpu as pltpu
from jax.experimental.pallas import tpu_sc as plsc

_B = 512


def _sc_gather_call(table, idx):
    N, F = table.shape
    assert F == 128
    E = idx.shape[0]
    info = plsc.get_sparse_core_info()
    NW = info.num_cores * info.num_subcores
    per_w = E // NW
    assert E % NW == 0 and per_w % 8 == 0
    CH = 512
    nfull = per_w // CH
    tail = per_w - nfull * CH
    tail_full = tail // 128
    tail_rem = tail % 128
    assert tail_rem % 8 == 0
    mesh = plsc.VectorSubcoreMesh(core_axis_name="c", subcore_axis_name="s")

    @functools.partial(
        pl.kernel, mesh=mesh,
        out_type=jax.ShapeDtypeStruct((E, F), jnp.float32),
        scratch_types=[
            pltpu.VMEM((CH,), jnp.int32),
            pltpu.VMEM((CH, F), jnp.float32),
            pltpu.SemaphoreType.DMA,
        ],
    )
    def gk(table_hbm, idx_hbm, out_hbm, idx_v, rows_v, sem):
        wid = lax.axis_index("s") * info.num_cores + lax.axis_index("c")
        base = wid * per_w

        def do_chunk(off, count):
            pltpu.sync_copy(idx_hbm.at[pl.ds(off, count)],
                            idx_v.at[pl.ds(0, count)])
            nsub = count // 128
            rem = count % 128
            cps = []
            for b in range(nsub):
                cps.append(pltpu.async_copy(
                    table_hbm.at[idx_v.at[pl.ds(b * 128, 128)]],
                    rows_v.at[pl.ds(b * 128, 128)], sem))
            if rem:
                cps.append(pltpu.async_copy(
                    table_hbm.at[idx_v.at[pl.ds(nsub * 128, rem)]],
                    rows_v.at[pl.ds(nsub * 128, rem)], sem))
            for c in cps:
                c.wait()
            pltpu.sync_copy(rows_v.at[pl.ds(0, count)],
                            out_hbm.at[pl.ds(off, count)])

        if nfull:
            def body(g, _):
                do_chunk(base + g * CH, CH)
                return _
            lax.fori_loop(0, nfull, body, None)
        if tail:
            do_chunk(base + nfull * CH, tail)

    return gk(table, idx)


def _edge_layer_call(geomT, gsv, gq, A1, b1, WksT, W3T):
    E = geomT.shape[1]
    B = _B if E % _B == 0 else E
    grid = E // B

    def body(geom_ref, gsv_ref, gq_ref, a1_ref, b1_ref, wks_ref,
             w3_ref, ms_ref, lg_ref):
        geom = geom_ref[...]
        r = geom[0:1]
        ew = geom[1:2]
        rhat = geom[2:5]
        a1 = a1_ref[...]
        h = jnp.maximum(a1[:, 0:1] * r + a1[:, 1:2] * ew + b1_ref[...], 0.0)
        gsvb = gsv_ref[...].T
        gs = gsvb[0:16]
        gV = gsvb[16:64]
        gV3 = gV.reshape(16, 3, B)
        vdotr = jnp.sum(gV3 * rhat[None, :, :], axis=1)
        g32 = jnp.concatenate([gs, vdotr], axis=0)
        U1 = (h[:, None, :] * g32[None, :, :]).reshape(512, B)
        kv = jnp.dot(wks_ref[...], U1, preferred_element_type=jnp.float32)
        k_s = kv[0:16]
        v_coef = kv[16:32]
        U2 = (h[:, None, :] * gV[None, :, :]).reshape(768, B)
        vv = jnp.dot(w3_ref[...], U2, preferred_element_type=jnp.float32)
        vvec = (vv.reshape(16, 3, B)
                + v_coef[:, None, :] * rhat[None, :, :]).reshape(48, B)
        gq = gq_ref[...].T
        logit = (jnp.sum(gq[0:16] * k_s, axis=0, keepdims=True)
                 + jnp.sum(gq[16:64] * vvec, axis=0, keepdims=True)) * 0.25
        ms_ref[...] = jnp.concatenate([k_s, vvec], axis=0).T
        lg_ref[...] = logit

    ebs = lambda rows: pl.BlockSpec((rows, B), lambda i: (0, i))
    rbs = lambda cols: pl.BlockSpec((B, cols), lambda i: (i, 0))
    wbs = lambda shape: pl.BlockSpec(shape, lambda i: (0, 0))
    return pl.pallas_call(
        body,
        grid=(grid,),
        in_specs=[ebs(8), rbs(128), rbs(128), wbs((16, 2)),
                  wbs((16, 1)), wbs((32, 512)), wbs((48, 768))],
        out_specs=[rbs(64), ebs(1)],
        out_shape=[jax.ShapeDtypeStruct((E, 64), jnp.float32),
                   jax.ShapeDtypeStruct((1, E), jnp.float32)],
        interpret=False,
    )(geomT, gsv, gq, A1, b1, WksT, W3T)


def _edge_final_call(geomT, gsv, gq, A1, b1, W0fT, W1fT):
    E = geomT.shape[1]
    B = _B if E % _B == 0 else E
    grid = E // B

    def body(geom_ref, gsv_ref, gq_ref, a1_ref, b1_ref, w0_ref,
             w1_ref, vv_ref, lg_ref):
        geom = geom_ref[...]
        r = geom[0:1]
        ew = geom[1:2]
        rhat = geom[2:5]
        a1 = a1_ref[...]
        h = jnp.maximum(a1[:, 0:1] * r + a1[:, 1:2] * ew + b1_ref[...], 0.0)
        gsvb = gsv_ref[...].T
        gs = gsvb[0:16]
        gV = gsvb[16:64]
        U1 = (h[:, None, :] * gs[None, :, :]).reshape(256, B)
        coef = jnp.dot(w0_ref[...], U1, preferred_element_type=jnp.float32)
        U2 = (h[:, None, :] * gV[None, :, :]).reshape(768, B)
        vv = jnp.dot(w1_ref[...], U2, preferred_element_type=jnp.float32)
        vvec = (vv.reshape(2, 3, B)
                + coef[:, None, :] * rhat[None, :, :]).reshape(6, B)
        gq = gq_ref[...].T
        logit = jnp.sum(gq[0:6] * vvec, axis=0, keepdims=True) * (1.0 / jnp.sqrt(2.0))
        vv_ref[...] = vvec.T
        lg_ref[...] = logit

    ebs = lambda rows: pl.BlockSpec((rows, B), lambda i: (0, i))
    rbs = lambda cols: pl.BlockSpec((B, cols), lambda i: (i, 0))
    wbs = lambda shape: pl.BlockSpec(shape, lambda i: (0, 0))
    return pl.pallas_call(
        body,
        grid=(grid,),
        in_specs=[ebs(8), rbs(128), rbs(128), wbs((16, 2)),
                  wbs((16, 1)), wbs((2, 256)), wbs((6, 768))],
        out_specs=[rbs(6), ebs(1)],
        out_shape=[jax.ShapeDtypeStruct((E, 6), jnp.float32),
                   jax.ShapeDtypeStruct((1, E), jnp.float32)],
        interpret=False,
    )(geomT, gsv, gq, A1, b1, W0fT, W1fT)


def _prep_layer_weights(Wr2_l):
    Wp = Wr2_l.reshape(16, 4, 16, 16)
    ks_s = Wp[:, 0].transpose(0, 2, 1)
    ks_r = Wp[:, 1].transpose(0, 2, 1)
    vc_s = Wp[:, 2].transpose(0, 2, 1)
    T = jnp.zeros((16, 32, 32), jnp.float32)
    T = T.at[:, :16, :16].set(ks_s)
    T = T.at[:, 16:, :16].set(ks_r)
    T = T.at[:, :16, 16:].set(vc_s)
    WksT = T.reshape(512, 32).T
    eye3 = jnp.eye(3, dtype=jnp.float32)
    T3 = jnp.einsum('hij,de->hjdie', Wp[:, 3], eye3)
    W3T = T3.reshape(768, 48).T
    return WksT, W3T


def _prep_final_weights(Wr2f):
    Wpf = Wr2f.reshape(16, 2, 2, 16)
    W0fT = Wpf[:, 0].transpose(1, 0, 2).reshape(2, 256)
    eye3 = jnp.eye(3, dtype=jnp.float32)
    T3 = jnp.einsum('hoc,de->hcdoe', Wpf[:, 1], eye3)
    W1fT = T3.reshape(768, 6).T
    return W0fT, W1fT


def kernel(pos, vel, edge_index, edge_w, Wemb, Wse, Wr1, br1, Wr2, Wqs, Wqv,
           Wss, Wsv, Wr1f, br1f, Wr2f, Wqf, Wof):
    N = pos.shape[0]
    E = edge_index.shape[1]
    src = edge_index[0]
    dst = edge_index[1]

    posp = jnp.pad(pos, ((0, 0), (0, 125)))
    ps = _sc_gather_call(posp, src)[:, :3]
    pd = _sc_gather_call(posp, dst)[:, :3]
    rel = pd - ps
    rn = jnp.linalg.norm(rel, axis=-1, keepdims=True) + 1e-8
    rhat = rel / rn
    geomT = jnp.concatenate(
        [rn.T, edge_w.T, rhat.T, jnp.zeros((3, E), jnp.float32)], axis=0)

    sp = jnp.linalg.norm(vel, axis=-1, keepdims=True)
    s = sp * Wse[0][None, :]
    V = vel[:, None, :] * Wemb[0][None, :, None]

    L = Wr1.shape[0]
    for l in range(L):
        WksT, W3T = _prep_layer_weights(Wr2[l])
        A1 = Wr1[l].T
        b1 = br1[l][:, None]
        q_s = s @ Wqs[l]
        q_v = jnp.einsum('ncd,ci->nid', V, Wqv[l]).reshape(N, 48)
        Vf = V.reshape(N, 48)
        gsv = _sc_gather_call(
            jnp.pad(jnp.concatenate([s, Vf], axis=-1), ((0, 0), (0, 64))), src)
        gq = _sc_gather_call(
            jnp.pad(jnp.concatenate([q_s, q_v], axis=-1), ((0, 0), (0, 64))), dst)
        msg, lgT = _edge_layer_call(geomT, gsv, gq, A1, b1, WksT, W3T)
        logit = lgT[0]
        m = jax.ops.segment_max(logit, dst, num_segments=N)
        ex = jnp.exp(logit - m[dst])
        den = jax.ops.segment_sum(ex, dst, num_segments=N) + 1e-9
        num = jax.ops.segment_sum(ex[:, None] * msg, dst, num_segments=N)
        agg = num / den[:, None]
        agg_s = agg[:, :16]
        agg_v = agg[:, 16:]
        s = jax.nn.relu(s + agg_s @ Wss[l])
        V = V + jnp.einsum('ncd,ci->nid', agg_v.reshape(N, 16, 3), Wsv[l])
        vn = jnp.linalg.norm(V, axis=-1, keepdims=True) + 1e-8
        V = V * jax.nn.sigmoid(vn)

    W0fT, W1fT = _prep_final_weights(Wr2f)
    A1f = Wr1f.T
    b1f = br1f[:, None]
    q_vf = jnp.einsum('ncd,co->nod', V, Wqf).reshape(N, 6)
    gsvf = _sc_gather_call(
        jnp.pad(jnp.concatenate([s, V.reshape(N, 48)], axis=-1),
                ((0, 0), (0, 64))), src)
    gqf = _sc_gather_call(jnp.pad(q_vf, ((0, 0), (0, 122))), dst)
    vvf, lgT = _edge_final_call(geomT, gsvf, gqf, A1f, b1f, W0fT, W1fT)
    logit = lgT[0]
    m = jax.ops.segment_max(logit, dst, num_segments=N)
    exf = jnp.exp(logit - m[dst])
    den = jax.ops.segment_sum(exf, dst, num_segments=N) + 1e-9
    num = jax.ops.segment_sum(exf[:, None] * vvf, dst, num_segments=N)
    agg = num / den[:, None]
    out = jnp.einsum('nod,op->npd', agg.reshape(N, 2, 3), Wof)
    return out


# speedup vs baseline: 12.6743x; 1.0317x over previous
import functools

import jax
import jax.numpy as jnp
from jax import lax
from jax.experimental import pallas as pl
from jax.experimental.pallas import tpu as pltpu
from jax.experimental.pallas import tpu_sc as plsc

_B = 512


def _sc_gather_call(table, idx):
    N, F = table.shape
    assert F == 128
    E = idx.shape[0]
    info = plsc.get_sparse_core_info()
    NW = info.num_cores * info.num_subcores
    per_w = E // NW
    assert E % NW == 0 and per_w % 8 == 0
    CH = 512
    nfull = per_w // CH
    tail = per_w - nfull * CH
    tail_full = tail // 128
    tail_rem = tail % 128
    assert tail_rem % 8 == 0
    mesh = plsc.VectorSubcoreMesh(core_axis_name="c", subcore_axis_name="s")

    @functools.partial(
        pl.kernel, mesh=mesh,
        out_type=jax.ShapeDtypeStruct((E, F), jnp.float32),
        scratch_types=[
            pltpu.VMEM((CH,), jnp.int32),
            pltpu.VMEM((CH, F), jnp.float32),
            pltpu.SemaphoreType.DMA,
        ],
    )
    def gk(table_hbm, idx_hbm, out_hbm, idx_v, rows_v, sem):
        wid = lax.axis_index("s") * info.num_cores + lax.axis_index("c")
        base = wid * per_w

        def do_chunk(off, count):
            pltpu.sync_copy(idx_hbm.at[pl.ds(off, count)],
                            idx_v.at[pl.ds(0, count)])
            nsub = count // 128
            rem = count % 128
            cps = []
            for b in range(nsub):
                cps.append(pltpu.async_copy(
                    table_hbm.at[idx_v.at[pl.ds(b * 128, 128)]],
                    rows_v.at[pl.ds(b * 128, 128)], sem))
            if rem:
                cps.append(pltpu.async_copy(
                    table_hbm.at[idx_v.at[pl.ds(nsub * 128, rem)]],
                    rows_v.at[pl.ds(nsub * 128, rem)], sem))
            for c in cps:
                c.wait()
            pltpu.sync_copy(rows_v.at[pl.ds(0, count)],
                            out_hbm.at[pl.ds(off, count)])

        if nfull:
            def body(g, _):
                do_chunk(base + g * CH, CH)
                return _
            lax.fori_loop(0, nfull, body, None)
        if tail:
            do_chunk(base + nfull * CH, tail)

    return gk(table, idx)


def _edge_layer_call(geomT, gsv, gq, A1, b1, WksT, W3T):
    E = geomT.shape[1]
    B = _B if E % _B == 0 else E
    grid = E // B

    def body(geom_ref, gsv_ref, gq_ref, a1_ref, b1_ref, wks_ref,
             w3_ref, ms_ref, lg_ref):
        geom = geom_ref[...]
        r = geom[0:1]
        ew = geom[1:2]
        rhat = geom[2:5]
        a1 = a1_ref[...]
        h = jnp.maximum(a1[:, 0:1] * r + a1[:, 1:2] * ew + b1_ref[...], 0.0)
        gsvb = gsv_ref[...].T
        gs = gsvb[0:16]
        gV = gsvb[16:64]
        gV3 = gV.reshape(16, 3, B)
        vdotr = jnp.sum(gV3 * rhat[None, :, :], axis=1)
        g32 = jnp.concatenate([gs, vdotr], axis=0)
        U1 = (h[:, None, :] * g32[None, :, :]).reshape(512, B)
        kv = jnp.dot(wks_ref[...], U1, preferred_element_type=jnp.float32)
        k_s = kv[0:16]
        v_coef = kv[16:32]
        U2 = (h[:, None, :] * gV[None, :, :]).reshape(768, B)
        vv = jnp.dot(w3_ref[...], U2, preferred_element_type=jnp.float32)
        vvec = (vv.reshape(16, 3, B)
                + v_coef[:, None, :] * rhat[None, :, :]).reshape(48, B)
        gq = gq_ref[...].T
        logit = (jnp.sum(gq[0:16] * k_s, axis=0, keepdims=True)
                 + jnp.sum(gq[16:64] * vvec, axis=0, keepdims=True)) * 0.25
        one = jnp.ones((1, B), jnp.float32)
        ms_ref[...] = jnp.concatenate([k_s, vvec, one], axis=0).T
        lg_ref[...] = logit

    ebs = lambda rows: pl.BlockSpec((rows, B), lambda i: (0, i))
    rbs = lambda cols: pl.BlockSpec((B, cols), lambda i: (i, 0))
    wbs = lambda shape: pl.BlockSpec(shape, lambda i: (0, 0))
    return pl.pallas_call(
        body,
        grid=(grid,),
        in_specs=[ebs(8), rbs(128), rbs(128), wbs((16, 2)),
                  wbs((16, 1)), wbs((32, 512)), wbs((48, 768))],
        out_specs=[rbs(65), ebs(1)],
        out_shape=[jax.ShapeDtypeStruct((E, 65), jnp.float32),
                   jax.ShapeDtypeStruct((1, E), jnp.float32)],
        interpret=False,
    )(geomT, gsv, gq, A1, b1, WksT, W3T)


def _edge_final_call(geomT, gsv, gq, A1, b1, W0fT, W1fT):
    E = geomT.shape[1]
    B = _B if E % _B == 0 else E
    grid = E // B

    def body(geom_ref, gsv_ref, gq_ref, a1_ref, b1_ref, w0_ref,
             w1_ref, vv_ref, lg_ref):
        geom = geom_ref[...]
        r = geom[0:1]
        ew = geom[1:2]
        rhat = geom[2:5]
        a1 = a1_ref[...]
        h = jnp.maximum(a1[:, 0:1] * r + a1[:, 1:2] * ew + b1_ref[...], 0.0)
        gsvb = gsv_ref[...].T
        gs = gsvb[0:16]
        gV = gsvb[16:64]
        U1 = (h[:, None, :] * gs[None, :, :]).reshape(256, B)
        coef = jnp.dot(w0_ref[...], U1, preferred_element_type=jnp.float32)
        U2 = (h[:, None, :] * gV[None, :, :]).reshape(768, B)
        vv = jnp.dot(w1_ref[...], U2, preferred_element_type=jnp.float32)
        vvec = (vv.reshape(2, 3, B)
                + coef[:, None, :] * rhat[None, :, :]).reshape(6, B)
        gq = gq_ref[...].T
        logit = jnp.sum(gq[0:6] * vvec, axis=0, keepdims=True) * (1.0 / jnp.sqrt(2.0))
        one = jnp.ones((1, B), jnp.float32)
        vv_ref[...] = jnp.concatenate([vvec, one], axis=0).T
        lg_ref[...] = logit

    ebs = lambda rows: pl.BlockSpec((rows, B), lambda i: (0, i))
    rbs = lambda cols: pl.BlockSpec((B, cols), lambda i: (i, 0))
    wbs = lambda shape: pl.BlockSpec(shape, lambda i: (0, 0))
    return pl.pallas_call(
        body,
        grid=(grid,),
        in_specs=[ebs(8), rbs(128), rbs(128), wbs((16, 2)),
                  wbs((16, 1)), wbs((2, 256)), wbs((6, 768))],
        out_specs=[rbs(7), ebs(1)],
        out_shape=[jax.ShapeDtypeStruct((E, 7), jnp.float32),
                   jax.ShapeDtypeStruct((1, E), jnp.float32)],
        interpret=False,
    )(geomT, gsv, gq, A1, b1, W0fT, W1fT)


def _prep_layer_weights(Wr2_l):
    Wp = Wr2_l.reshape(16, 4, 16, 16)
    ks_s = Wp[:, 0].transpose(0, 2, 1)
    ks_r = Wp[:, 1].transpose(0, 2, 1)
    vc_s = Wp[:, 2].transpose(0, 2, 1)
    T = jnp.zeros((16, 32, 32), jnp.float32)
    T = T.at[:, :16, :16].set(ks_s)
    T = T.at[:, 16:, :16].set(ks_r)
    T = T.at[:, :16, 16:].set(vc_s)
    WksT = T.reshape(512, 32).T
    eye3 = jnp.eye(3, dtype=jnp.float32)
    T3 = jnp.einsum('hij,de->hjdie', Wp[:, 3], eye3)
    W3T = T3.reshape(768, 48).T
    return WksT, W3T


def _prep_final_weights(Wr2f):
    Wpf = Wr2f.reshape(16, 2, 2, 16)
    W0fT = Wpf[:, 0].transpose(1, 0, 2).reshape(2, 256)
    eye3 = jnp.eye(3, dtype=jnp.float32)
    T3 = jnp.einsum('hoc,de->hcdoe', Wpf[:, 1], eye3)
    W1fT = T3.reshape(768, 6).T
    return W0fT, W1fT


def kernel(pos, vel, edge_index, edge_w, Wemb, Wse, Wr1, br1, Wr2, Wqs, Wqv,
           Wss, Wsv, Wr1f, br1f, Wr2f, Wqf, Wof):
    N = pos.shape[0]
    E = edge_index.shape[1]
    src = edge_index[0]
    dst = edge_index[1]

    posp = jnp.pad(pos, ((0, 0), (0, 125)))
    ps = _sc_gather_call(posp, src)[:, :3]
    pd = _sc_gather_call(posp, dst)[:, :3]
    rel = pd - ps
    rn = jnp.linalg.norm(rel, axis=-1, keepdims=True) + 1e-8
    rhat = rel / rn
    geomT = jnp.concatenate(
        [rn.T, edge_w.T, rhat.T, jnp.zeros((3, E), jnp.float32)], axis=0)

    sp = jnp.linalg.norm(vel, axis=-1, keepdims=True)
    s = sp * Wse[0][None, :]
    V = vel[:, None, :] * Wemb[0][None, :, None]

    L = Wr1.shape[0]
    for l in range(L):
        WksT, W3T = _prep_layer_weights(Wr2[l])
        A1 = Wr1[l].T
        b1 = br1[l][:, None]
        q_s = s @ Wqs[l]
        q_v = jnp.einsum('ncd,ci->nid', V, Wqv[l]).reshape(N, 48)
        Vf = V.reshape(N, 48)
        gsv = _sc_gather_call(
            jnp.pad(jnp.concatenate([s, Vf], axis=-1), ((0, 0), (0, 64))), src)
        gq = _sc_gather_call(
            jnp.pad(jnp.concatenate([q_s, q_v], axis=-1), ((0, 0), (0, 64))), dst)
        msg, lgT = _edge_layer_call(geomT, gsv, gq, A1, b1, WksT, W3T)
        logit = lgT[0]
        m = jax.ops.segment_max(logit, dst, num_segments=N)
        ex = jnp.exp(logit - m[dst])
        nd = jax.ops.segment_sum(ex[:, None] * msg, dst, num_segments=N)
        agg = nd[:, :64] / (nd[:, 64:65] + 1e-9)
        agg_s = agg[:, :16]
        agg_v = agg[:, 16:]
        s = jax.nn.relu(s + agg_s @ Wss[l])
        V = V + jnp.einsum('ncd,ci->nid', agg_v.reshape(N, 16, 3), Wsv[l])
        vn = jnp.linalg.norm(V, axis=-1, keepdims=True) + 1e-8
        V = V * jax.nn.sigmoid(vn)

    W0fT, W1fT = _prep_final_weights(Wr2f)
    A1f = Wr1f.T
    b1f = br1f[:, None]
    q_vf = jnp.einsum('ncd,co->nod', V, Wqf).reshape(N, 6)
    gsvf = _sc_gather_call(
        jnp.pad(jnp.concatenate([s, V.reshape(N, 48)], axis=-1),
                ((0, 0), (0, 64))), src)
    gqf = _sc_gather_call(jnp.pad(q_vf, ((0, 0), (0, 122))), dst)
    vvf, lgT = _edge_final_call(geomT, gsvf, gqf, A1f, b1f, W0fT, W1fT)
    logit = lgT[0]
    m = jax.ops.segment_max(logit, dst, num_segments=N)
    exf = jnp.exp(logit - m[dst])
    nd = jax.ops.segment_sum(exf[:, None] * vvf, dst, num_segments=N)
    agg = nd[:, :6] / (nd[:, 6:7] + 1e-9)
    out = jnp.einsum('nod,op->npd', agg.reshape(N, 2, 3), Wof)
    return out
